# baseline (speedup 1.0000x reference)
import functools

import jax
import jax.numpy as jnp
from jax import lax
from jax.experimental import pallas as pl
from jax.experimental.pallas import tpu as pltpu
from jax.experimental.pallas import tpu_sc as plsc

N = 10000
E = 320000
D = 128
DH = D // 2
NC = 2
NS = 16
NP = 10240
CH = 128
EP = 327680
CPS = 4
CHR = CPS * CH
TROWS = EP // CH // NS
SLABR = 80
NSLAB = TROWS // SLABR
SPS = SLABR // CPS
NBUF = 2
RPT = NP // NS

_mesh = plsc.VectorSubcoreMesh(core_axis_name="c", subcore_axis_name="s")
_sc_params = pltpu.CompilerParams(use_tc_tiling_on_sc=False)


def _agg_body(with_deg, *refs):
    sems = refs[-2 * NBUF:]
    if with_deg:
        (table_hbm, src_hbm, srcn_hbm, dst_hbm, zrows_hbm, zcol_hbm,
         ones_hbm,
         psum_hbm, pdeg_hbm,
         acc_sh, deg_sh, sidx_v, didx_v, rows_v, ones_v) = refs[:-2 * NBUF]
    else:
        (table_hbm, src_hbm, srcn_hbm, dst_hbm, zrows_hbm,
         psum_hbm,
         acc_sh, sidx_v, didx_v, rows_v) = refs[:-2 * NBUF]
    gsems = sems[:NBUF]
    ssems = sems[NBUF:]

    cid = lax.axis_index("c")
    sid = lax.axis_index("s")
    r0 = sid * RPT
    core0 = cid == 0

    pltpu.sync_copy(zrows_hbm, acc_sh.at[pl.ds(r0, RPT)])
    if with_deg:
        @pl.when(core0)
        def _():
            pltpu.sync_copy(zcol_hbm, deg_sh.at[pl.ds(r0, RPT)])
            pltpu.sync_copy(ones_hbm, ones_v)
    plsc.subcore_barrier()

    ebase = sid * TROWS * CH

    def fire_gather(t, b):
        idx = sidx_v.at[pl.ds(CHR * t, CHR)]
        pltpu.async_copy(table_hbm.at[idx], rows_v.at[b], gsems[b])

    def scatter_parts(t, b):
        for k in range(CPS):
            yield (rows_v.at[b, pl.ds(k * CH, CH)], didx_v.at[CPS * t + k])

    for s in range(NSLAB):
        @pl.when(core0)
        def _():
            pltpu.sync_copy(src_hbm.at[pl.ds(ebase + s * SLABR * CH,
                                             SLABR * CH)], sidx_v)

        @pl.when(jnp.logical_not(core0))
        def _():
            pltpu.sync_copy(srcn_hbm.at[pl.ds(ebase + s * SLABR * CH,
                                              SLABR * CH)], sidx_v)

        pltpu.sync_copy(
            dst_hbm.at[pl.ds(sid * TROWS + s * SLABR, SLABR)], didx_v)
        for b in range(NBUF):
            fire_gather(b, b)

        def group(g, carry):
            for b in range(NBUF):
                t = NBUF * g + b
                idx = sidx_v.at[pl.ds(CHR * t, CHR)]
                pltpu.make_async_copy(table_hbm.at[idx],
                                      rows_v.at[b], gsems[b]).wait()
                for rsrc, didx in scatter_parts(t, b):
                    pltpu.async_copy(rsrc, acc_sh.at[didx],
                                     ssems[b], add=True)
                    if with_deg:
                        @pl.when(core0)
                        def _():
                            pltpu.async_copy(ones_v, deg_sh.at[didx],
                                             ssems[b], add=True)
            for b in range(NBUF):
                t = NBUF * g + b
                t2 = t + NBUF
                for rsrc, didx in scatter_parts(t, b):
                    pltpu.make_async_copy(rsrc, acc_sh.at[didx],
                                          ssems[b]).wait()
                    if with_deg:
                        @pl.when(core0)
                        def _():
                            pltpu.make_async_copy(ones_v, deg_sh.at[didx],
                                                  ssems[b]).wait()

                @pl.when(t2 < SPS)
                def _():
                    fire_gather(t2, b)
            return carry

        lax.fori_loop(0, SPS // NBUF, group, 0)

    plsc.subcore_barrier()
    pltpu.sync_copy(acc_sh.at[pl.ds(r0, RPT)], psum_hbm.at[cid, pl.ds(r0, RPT)])
    if with_deg:
        @pl.when(core0)
        def _():
            pltpu.sync_copy(deg_sh.at[pl.ds(r0, RPT)],
                            pdeg_hbm.at[pl.ds(r0, RPT)])


def _sc_agg_deg(table, src, srcn, dst, zrows, zcol, ones):
    f = pl.kernel(
        functools.partial(_agg_body, True),
        mesh=_mesh,
        out_type=[
            jax.ShapeDtypeStruct((NC, NP, DH), jnp.float32),
            jax.ShapeDtypeStruct((NP,), jnp.float32),
        ],
        scratch_types=[
            pltpu.VMEM_SHARED((NP, DH), jnp.float32),
            pltpu.VMEM_SHARED((NP,), jnp.float32),
            pltpu.VMEM((SLABR * CH,), jnp.int32),
            pltpu.VMEM((SLABR, CH), jnp.int32),
            pltpu.VMEM((NBUF, CHR, DH), jnp.float32),
            pltpu.VMEM((CH,), jnp.float32),
        ] + [pltpu.SemaphoreType.DMA] * (2 * NBUF),
        compiler_params=_sc_params,
    )
    return f(table, src, srcn, dst, zrows, zcol, ones)


def _sc_agg(table, src, srcn, dst, zrows):
    f = pl.kernel(
        functools.partial(_agg_body, False),
        mesh=_mesh,
        out_type=jax.ShapeDtypeStruct((NC, NP, DH), jnp.float32),
        scratch_types=[
            pltpu.VMEM_SHARED((NP, DH), jnp.float32),
            pltpu.VMEM((SLABR * CH,), jnp.int32),
            pltpu.VMEM((SLABR, CH), jnp.int32),
            pltpu.VMEM((NBUF, CHR, DH), jnp.float32),
        ] + [pltpu.SemaphoreType.DMA] * (2 * NBUF),
        compiler_params=_sc_params,
    )
    return f(table, src, srcn, dst, zrows)


_DOT = functools.partial(
    lax.dot_general,
    dimension_numbers=(((1,), (0,)), ((), ())),
    preferred_element_type=jnp.float32,
    precision=lax.Precision.HIGHEST,
)

_R = 1000


def _root1_body(x_ref, wr_ref, b_ref, o_ref):
    o_ref[...] = _DOT(x_ref[...], wr_ref[...]) + b_ref[...]


def _root1(x, wr, b):
    return pl.pallas_call(
        _root1_body,
        grid=(N // _R,),
        in_specs=[
            pl.BlockSpec((_R, D), lambda i: (i, 0)),
            pl.BlockSpec((D, D), lambda i: (0, 0)),
            pl.BlockSpec((1, D), lambda i: (0, 0)),
        ],
        out_specs=pl.BlockSpec((_R, D), lambda i: (i, 0)),
        out_shape=jax.ShapeDtypeStruct((N, D), jnp.float32),
    )(x, wr, b)


def _dense1_body(ps_ref, dg_ref, r_ref, wl_ref, o_ref):
    inv = 1.0 / jnp.maximum(dg_ref[...], 1.0)
    wl = wl_ref[...]
    h1 = jnp.maximum(_DOT(ps_ref[0] * inv, wl[:DH])
                     + _DOT(ps_ref[1] * inv, wl[DH:]) + r_ref[...], 0.0)
    o_ref[0] = h1[:, :DH]
    o_ref[1] = h1[:, DH:]


def _dense1(psum, pdeg2, r1, wl):
    return pl.pallas_call(
        _dense1_body,
        grid=(N // _R,),
        in_specs=[
            pl.BlockSpec((NC, _R, DH), lambda i: (0, i, 0)),
            pl.BlockSpec((_R, 1), lambda i: (i, 0)),
            pl.BlockSpec((_R, D), lambda i: (i, 0)),
            pl.BlockSpec((D, D), lambda i: (0, 0)),
        ],
        out_specs=pl.BlockSpec((NC, _R, DH), lambda i: (0, i, 0)),
        out_shape=jax.ShapeDtypeStruct((NC, N, DH), jnp.float32),
    )(psum, pdeg2, r1, wl)


def _root2_body(hs_ref, wr_ref, b_ref, wa_ref, r_ref, o1_ref):
    h1l = hs_ref[0]
    h1r = hs_ref[1]
    wr = wr_ref[...]
    wa = wa_ref[...]
    r_ref[...] = _DOT(h1l, wr[:DH]) + _DOT(h1r, wr[DH:]) + b_ref[...]
    o1_ref[...] = _DOT(h1l, wa[:DH]) + _DOT(h1r, wa[DH:])


def _root2(hs, wr, b, wa):
    return pl.pallas_call(
        _root2_body,
        grid=(N // _R,),
        in_specs=[
            pl.BlockSpec((NC, _R, DH), lambda i: (0, i, 0)),
            pl.BlockSpec((D, D), lambda i: (0, 0)),
            pl.BlockSpec((1, D), lambda i: (0, 0)),
            pl.BlockSpec((D, 1), lambda i: (0, 0)),
        ],
        out_specs=[
            pl.BlockSpec((_R, D), lambda i: (i, 0)),
            pl.BlockSpec((_R, 1), lambda i: (i, 0)),
        ],
        out_shape=[
            jax.ShapeDtypeStruct((N, D), jnp.float32),
            jax.ShapeDtypeStruct((N, 1), jnp.float32),
        ],
    )(hs, wr, b, wa)


def _dense2_body(ps_ref, dg_ref, r_ref, o1_ref, wl_ref, wb_ref, bro_ref,
                 o_ref):
    inv = 1.0 / jnp.maximum(dg_ref[...], 1.0)
    wl = wl_ref[...]
    h2 = jnp.maximum(_DOT(ps_ref[0] * inv, wl[:DH])
                     + _DOT(ps_ref[1] * inv, wl[DH:]) + r_ref[...], 0.0)
    o_ref[...] = o1_ref[...] + _DOT(h2, wb_ref[...]) + bro_ref[...]


def _dense2(psum, pdeg2, r2, o1, wl, wb, bro):
    return pl.pallas_call(
        _dense2_body,
        grid=(N // _R,),
        in_specs=[
            pl.BlockSpec((NC, _R, DH), lambda i: (0, i, 0)),
            pl.BlockSpec((_R, 1), lambda i: (i, 0)),
            pl.BlockSpec((_R, D), lambda i: (i, 0)),
            pl.BlockSpec((_R, 1), lambda i: (i, 0)),
            pl.BlockSpec((D, D), lambda i: (0, 0)),
            pl.BlockSpec((D, 1), lambda i: (0, 0)),
            pl.BlockSpec((1, 1), lambda i: (0, 0)),
        ],
        out_specs=pl.BlockSpec((_R, 1), lambda i: (i, 0)),
        out_shape=jax.ShapeDtypeStruct((N, 1), jnp.float32),
    )(psum, pdeg2, r2, o1, wl, wb, bro)


def kernel(x, edge_index, W1l, W1r, b1, W2l, W2r, b2, Wro, bro):
    src = edge_index[0]
    dst = edge_index[1]
    pad = jnp.arange(EP - E, dtype=jnp.int32)
    srcp = jnp.concatenate([src, pad % N])
    srcn = srcp + N
    dstp = jnp.concatenate([dst, N + pad % (NP - N)]).reshape(EP // CH, CH)
    zrows = jnp.zeros((RPT, DH), jnp.float32)
    zcol = jnp.zeros((RPT,), jnp.float32)
    ones = jnp.ones((CH,), jnp.float32)

    xs = jnp.concatenate([x[:, :DH], x[:, DH:]], axis=0)

    r1 = _root1(x, W1r, b1.reshape(1, D))
    psum1, pdeg = _sc_agg_deg(xs, srcp, srcn, dstp, zrows, zcol, ones)
    pdeg2 = pdeg[:N, None]
    h1s = _dense1(psum1, pdeg2, r1, W1l)
    r2, o1 = _root2(h1s, W2r, b2.reshape(1, D), Wro[:D])
    psum2 = _sc_agg(h1s.reshape(NC * N, DH), srcp, srcn, dstp, zrows)
    out = _dense2(psum2, pdeg2, r2, o1, W2l, Wro[D:], bro.reshape(1, 1))
    return out


# speedup vs baseline: 1.1428x; 1.1428x over previous
import functools

import jax
import jax.numpy as jnp
from jax import lax
from jax.experimental import pallas as pl
from jax.experimental.pallas import tpu as pltpu
from jax.experimental.pallas import tpu_sc as plsc

N = 10000
E = 320000
D = 128
DH = D // 2
NC = 2
NS = 16
NP = 10240
CH = 128
EP = 327680
CPS = 2
CHR = CPS * CH
TROWS = EP // CH // NS
SLABR = 80
NSLAB = TROWS // SLABR
SPS = SLABR // CPS
NBUF = 4
RPT = NP // NS

_mesh = plsc.VectorSubcoreMesh(core_axis_name="c", subcore_axis_name="s")
_sc_params = pltpu.CompilerParams(use_tc_tiling_on_sc=False)


def _agg_body(with_deg, *refs):
    if with_deg:
        (table_hbm, src_hbm, srcn_hbm, dst_hbm, zrows_hbm, zcol_hbm,
         ones_hbm,
         psum_hbm, pdeg_hbm,
         acc_sh, deg_sh, sidx_v, didx_v, rows_v, ones_v,
         g0, g1, g2, g3, s0, s1, s2, s3) = refs
    else:
        (table_hbm, src_hbm, srcn_hbm, dst_hbm, zrows_hbm,
         psum_hbm,
         acc_sh, sidx_v, didx_v, rows_v,
         g0, g1, g2, g3, s0, s1, s2, s3) = refs
    gsems = (g0, g1, g2, g3)
    ssems = (s0, s1, s2, s3)

    cid = lax.axis_index("c")
    sid = lax.axis_index("s")
    r0 = sid * RPT
    core0 = cid == 0

    pltpu.sync_copy(zrows_hbm, acc_sh.at[pl.ds(r0, RPT)])
    if with_deg:
        @pl.when(core0)
        def _():
            pltpu.sync_copy(zcol_hbm, deg_sh.at[pl.ds(r0, RPT)])
            pltpu.sync_copy(ones_hbm, ones_v)
    plsc.subcore_barrier()

    ebase = sid * TROWS * CH

    def fire_gather(t, b):
        idx = sidx_v.at[pl.ds(CHR * t, CHR)]
        pltpu.async_copy(table_hbm.at[idx], rows_v.at[b], gsems[b])

    def scatter_parts(t, b):
        for k in range(CPS):
            yield (rows_v.at[b, pl.ds(k * CH, CH)], didx_v.at[CPS * t + k])

    for s in range(NSLAB):
        @pl.when(core0)
        def _():
            pltpu.sync_copy(src_hbm.at[pl.ds(ebase + s * SLABR * CH,
                                             SLABR * CH)], sidx_v)

        @pl.when(jnp.logical_not(core0))
        def _():
            pltpu.sync_copy(srcn_hbm.at[pl.ds(ebase + s * SLABR * CH,
                                              SLABR * CH)], sidx_v)

        pltpu.sync_copy(
            dst_hbm.at[pl.ds(sid * TROWS + s * SLABR, SLABR)], didx_v)
        for b in range(NBUF):
            fire_gather(b, b)

        def group(g, carry):
            for b in range(NBUF):
                t = NBUF * g + b
                idx = sidx_v.at[pl.ds(CHR * t, CHR)]
                pltpu.make_async_copy(table_hbm.at[idx],
                                      rows_v.at[b], gsems[b]).wait()
                for rsrc, didx in scatter_parts(t, b):
                    pltpu.async_copy(rsrc, acc_sh.at[didx],
                                     ssems[b], add=True)
                    if with_deg:
                        @pl.when(core0)
                        def _():
                            pltpu.async_copy(ones_v, deg_sh.at[didx],
                                             ssems[b], add=True)
            for b in range(NBUF):
                t = NBUF * g + b
                t2 = t + NBUF
                for rsrc, didx in scatter_parts(t, b):
                    pltpu.make_async_copy(rsrc, acc_sh.at[didx],
                                          ssems[b]).wait()
                    if with_deg:
                        @pl.when(core0)
                        def _():
                            pltpu.make_async_copy(ones_v, deg_sh.at[didx],
                                                  ssems[b]).wait()

                @pl.when(t2 < SPS)
                def _():
                    fire_gather(t2, b)
            return carry

        lax.fori_loop(0, SPS // NBUF, group, 0)

    plsc.subcore_barrier()
    pltpu.sync_copy(acc_sh.at[pl.ds(r0, RPT)], psum_hbm.at[cid, pl.ds(r0, RPT)])
    if with_deg:
        @pl.when(core0)
        def _():
            pltpu.sync_copy(deg_sh.at[pl.ds(r0, RPT)],
                            pdeg_hbm.at[pl.ds(r0, RPT)])


def _sc_agg_deg(table, src, srcn, dst, zrows, zcol, ones):
    f = pl.kernel(
        functools.partial(_agg_body, True),
        mesh=_mesh,
        out_type=[
            jax.ShapeDtypeStruct((NC, NP, DH), jnp.float32),
            jax.ShapeDtypeStruct((NP,), jnp.float32),
        ],
        scratch_types=[
            pltpu.VMEM_SHARED((NP, DH), jnp.float32),
            pltpu.VMEM_SHARED((NP,), jnp.float32),
            pltpu.VMEM((SLABR * CH,), jnp.int32),
            pltpu.VMEM((SLABR, CH), jnp.int32),
            pltpu.VMEM((NBUF, CHR, DH), jnp.float32),
            pltpu.VMEM((CH,), jnp.float32),
        ] + [pltpu.SemaphoreType.DMA] * (2 * NBUF),
        compiler_params=_sc_params,
    )
    return f(table, src, srcn, dst, zrows, zcol, ones)


def _sc_agg(table, src, srcn, dst, zrows):
    f = pl.kernel(
        functools.partial(_agg_body, False),
        mesh=_mesh,
        out_type=jax.ShapeDtypeStruct((NC, NP, DH), jnp.float32),
        scratch_types=[
            pltpu.VMEM_SHARED((NP, DH), jnp.float32),
            pltpu.VMEM((SLABR * CH,), jnp.int32),
            pltpu.VMEM((SLABR, CH), jnp.int32),
            pltpu.VMEM((NBUF, CHR, DH), jnp.float32),
        ] + [pltpu.SemaphoreType.DMA] * (2 * NBUF),
        compiler_params=_sc_params,
    )
    return f(table, src, srcn, dst, zrows)


_DOT = functools.partial(
    lax.dot_general,
    dimension_numbers=(((1,), (0,)), ((), ())),
    preferred_element_type=jnp.float32,
    precision=lax.Precision.HIGHEST,
)

_R = 1000


def _root1_body(x_ref, wr_ref, b_ref, o_ref):
    o_ref[...] = _DOT(x_ref[...], wr_ref[...]) + b_ref[...]


def _root1(x, wr, b):
    return pl.pallas_call(
        _root1_body,
        grid=(N // _R,),
        in_specs=[
            pl.BlockSpec((_R, D), lambda i: (i, 0)),
            pl.BlockSpec((D, D), lambda i: (0, 0)),
            pl.BlockSpec((1, D), lambda i: (0, 0)),
        ],
        out_specs=pl.BlockSpec((_R, D), lambda i: (i, 0)),
        out_shape=jax.ShapeDtypeStruct((N, D), jnp.float32),
    )(x, wr, b)


def _dense1_body(ps_ref, dg_ref, r_ref, wl_ref, o_ref):
    inv = 1.0 / jnp.maximum(dg_ref[...], 1.0)
    wl = wl_ref[...]
    h1 = jnp.maximum(_DOT(ps_ref[0] * inv, wl[:DH])
                     + _DOT(ps_ref[1] * inv, wl[DH:]) + r_ref[...], 0.0)
    o_ref[0] = h1[:, :DH]
    o_ref[1] = h1[:, DH:]


def _dense1(psum, pdeg2, r1, wl):
    return pl.pallas_call(
        _dense1_body,
        grid=(N // _R,),
        in_specs=[
            pl.BlockSpec((NC, _R, DH), lambda i: (0, i, 0)),
            pl.BlockSpec((_R, 1), lambda i: (i, 0)),
            pl.BlockSpec((_R, D), lambda i: (i, 0)),
            pl.BlockSpec((D, D), lambda i: (0, 0)),
        ],
        out_specs=pl.BlockSpec((NC, _R, DH), lambda i: (0, i, 0)),
        out_shape=jax.ShapeDtypeStruct((NC, N, DH), jnp.float32),
    )(psum, pdeg2, r1, wl)


def _root2_body(hs_ref, wr_ref, b_ref, wa_ref, r_ref, o1_ref):
    h1l = hs_ref[0]
    h1r = hs_ref[1]
    wr = wr_ref[...]
    wa = wa_ref[...]
    r_ref[...] = _DOT(h1l, wr[:DH]) + _DOT(h1r, wr[DH:]) + b_ref[...]
    o1_ref[...] = _DOT(h1l, wa[:DH]) + _DOT(h1r, wa[DH:])


def _root2(hs, wr, b, wa):
    return pl.pallas_call(
        _root2_body,
        grid=(N // _R,),
        in_specs=[
            pl.BlockSpec((NC, _R, DH), lambda i: (0, i, 0)),
            pl.BlockSpec((D, D), lambda i: (0, 0)),
            pl.BlockSpec((1, D), lambda i: (0, 0)),
            pl.BlockSpec((D, 1), lambda i: (0, 0)),
        ],
        out_specs=[
            pl.BlockSpec((_R, D), lambda i: (i, 0)),
            pl.BlockSpec((_R, 1), lambda i: (i, 0)),
        ],
        out_shape=[
            jax.ShapeDtypeStruct((N, D), jnp.float32),
            jax.ShapeDtypeStruct((N, 1), jnp.float32),
        ],
    )(hs, wr, b, wa)


def _dense2_body(ps_ref, dg_ref, r_ref, o1_ref, wl_ref, wb_ref, bro_ref,
                 o_ref):
    inv = 1.0 / jnp.maximum(dg_ref[...], 1.0)
    wl = wl_ref[...]
    h2 = jnp.maximum(_DOT(ps_ref[0] * inv, wl[:DH])
                     + _DOT(ps_ref[1] * inv, wl[DH:]) + r_ref[...], 0.0)
    o_ref[...] = o1_ref[...] + _DOT(h2, wb_ref[...]) + bro_ref[...]


def _dense2(psum, pdeg2, r2, o1, wl, wb, bro):
    return pl.pallas_call(
        _dense2_body,
        grid=(N // _R,),
        in_specs=[
            pl.BlockSpec((NC, _R, DH), lambda i: (0, i, 0)),
            pl.BlockSpec((_R, 1), lambda i: (i, 0)),
            pl.BlockSpec((_R, D), lambda i: (i, 0)),
            pl.BlockSpec((_R, 1), lambda i: (i, 0)),
            pl.BlockSpec((D, D), lambda i: (0, 0)),
            pl.BlockSpec((D, 1), lambda i: (0, 0)),
            pl.BlockSpec((1, 1), lambda i: (0, 0)),
        ],
        out_specs=pl.BlockSpec((_R, 1), lambda i: (i, 0)),
        out_shape=jax.ShapeDtypeStruct((N, 1), jnp.float32),
    )(psum, pdeg2, r2, o1, wl, wb, bro)


def kernel(x, edge_index, W1l, W1r, b1, W2l, W2r, b2, Wro, bro):
    src = edge_index[0]
    dst = edge_index[1]
    pad = jnp.arange(EP - E, dtype=jnp.int32)
    srcp = jnp.concatenate([src, pad % N])
    srcn = srcp + N
    dstp = jnp.concatenate([dst, N + pad % (NP - N)]).reshape(EP // CH, CH)
    zrows = jnp.zeros((RPT, DH), jnp.float32)
    zcol = jnp.zeros((RPT,), jnp.float32)
    ones = jnp.ones((CH,), jnp.float32)

    xs = jnp.concatenate([x[:, :DH], x[:, DH:]], axis=0)

    r1 = _root1(x, W1r, b1.reshape(1, D))
    psum1, pdeg = _sc_agg_deg(xs, srcp, srcn, dstp, zrows, zcol, ones)
    pdeg2 = pdeg[:N, None]
    h1s = _dense1(psum1, pdeg2, r1, W1l)
    r2, o1 = _root2(h1s, W2r, b2.reshape(1, D), Wro[:D])
    psum2 = _sc_agg(h1s.reshape(NC * N, DH), srcp, srcn, dstp, zrows)
    out = _dense2(psum2, pdeg2, r2, o1, W2l, Wro[D:], bro.reshape(1, 1))
    return out
